# Initial kernel scaffold; baseline (speedup 1.0000x reference)
import jax
import jax.numpy as jnp
from jax.experimental import pallas as pl


def kernel(edge_index, emb, Wl0, bl0, Wr0, g0, b0, rm0, rv0, Wl1, bl1, Wr1, g1, b1, rm1, rv1):
    raise NotImplementedError("write your pallas kernel here")


# speedup vs baseline: 6.8778x; 6.8778x over previous
import dataclasses
import functools

import jax
import jax.numpy as jnp
from jax import lax
from jax.experimental import pallas as pl
from jax.experimental.pallas import tpu as pltpu
from jax.experimental.pallas import tpu_sc as plsc

N = 10000
E = 320000
D = 128
EPS = 1e-5

NC = 2
NS = 16
NW = NC * NS
CHUNK = 128
NCHUNKS = E // CHUNK
ITERS = -(-NCHUNKS // NW)
NPAD = 10240
RPT = NPAD // NS


def _sc_agg_build(with_deg: bool):
    out_type = [jax.ShapeDtypeStruct((NC, NPAD, D), jnp.float32)]
    if with_deg:
        out_type.append(jax.ShapeDtypeStruct((NW, 1, N), jnp.float32))

    scratch = [
        pltpu.VMEM((CHUNK,), jnp.int32),
        pltpu.VMEM((CHUNK,), jnp.int32),
        pltpu.VMEM((CHUNK, D), jnp.float32),
        pltpu.VMEM((1, N), jnp.float32),
        pltpu.VMEM_SHARED((NPAD, D), jnp.float32),
        pltpu.SemaphoreType.DMA,
    ]
    mesh = plsc.VectorSubcoreMesh(core_axis_name="c", subcore_axis_name="s")
    cp = pltpu.CompilerParams()
    if "needs_layout_passes" in pltpu.CompilerParams.__dataclass_fields__:
        cp = dataclasses.replace(cp, needs_layout_passes=False)

    @functools.partial(pl.kernel, out_type=tuple(out_type), mesh=mesh,
                       scratch_types=scratch, compiler_params=cp)
    def sc_agg(*refs):
        if with_deg:
            (src_hbm, dst_hbm, x_hbm, agg_out, deg_out,
             idx_s, idx_d, rows, deg_v, agg_sh, sem) = refs
        else:
            (src_hbm, dst_hbm, x_hbm, agg_out,
             idx_s, idx_d, rows, deg_v, agg_sh, sem) = refs

        cid = lax.axis_index("c")
        sid = lax.axis_index("s")
        wid = sid * NC + cid

        zero16 = jnp.zeros((16,), jnp.float32)
        one16 = jnp.full((16,), 1.0, jnp.float32)
        zero16i = jnp.zeros((16,), jnp.int32)

        @pl.loop(0, CHUNK)
        def _(i):
            for k in range(D // 16):
                rows[i, pl.ds(k * 16, 16)] = zero16

        base_r = sid * RPT
        for k in range(RPT // CHUNK):
            pltpu.sync_copy(rows, agg_sh.at[pl.ds(base_r + k * CHUNK, CHUNK)])

        if with_deg:
            @pl.loop(0, N, step=16)
            def _(i):
                deg_v[0, pl.ds(i, 16)] = zero16

        plsc.subcore_barrier()

        @pl.loop(0, ITERS)
        def _(j):
            chunk = j * NW + wid

            @pl.when(chunk < NCHUNKS)
            def _():
                base = chunk * CHUNK
                pltpu.sync_copy(src_hbm.at[pl.ds(base, CHUNK)], idx_s)
                pltpu.sync_copy(dst_hbm.at[pl.ds(base, CHUNK)], idx_d)
                pltpu.async_copy(x_hbm.at[idx_s], rows, sem).wait()
                pltpu.sync_copy(rows, agg_sh.at[idx_d], add=True)
                if with_deg:
                    for k in range(CHUNK // 16):
                        plsc.addupdate_scatter(
                            deg_v, [zero16i, idx_d[pl.ds(k * 16, 16)]], one16)

        plsc.subcore_barrier()

        pltpu.sync_copy(agg_sh.at[pl.ds(base_r, RPT)],
                        agg_out.at[cid, pl.ds(base_r, RPT)])
        if with_deg:
            pltpu.sync_copy(deg_v, deg_out.at[wid])

    return sc_agg


_sc_agg_deg = _sc_agg_build(True)
_sc_agg = _sc_agg_build(False)

BN_ROWS = 2000


def _dense_body(x_ref, a_ref, d_ref, wl_ref, wr_ref, s_ref, t_ref, o_ref):
    agg = a_ref[0] + a_ref[1]
    deg = jnp.sum(d_ref[...], axis=1, keepdims=True)
    mean = agg / jnp.maximum(deg, 1.0)
    h = jnp.dot(mean, wl_ref[...], preferred_element_type=jnp.float32,
                precision=lax.Precision.HIGHEST)
    h = h + jnp.dot(x_ref[...], wr_ref[...], preferred_element_type=jnp.float32,
                    precision=lax.Precision.HIGHEST)
    h = h * s_ref[...] + t_ref[...]
    o_ref[...] = x_ref[...] + jnp.maximum(h, 0.0)


def _dense(x, aggp, degT, WlT, WrT, s, t):
    grid = (N // BN_ROWS,)
    return pl.pallas_call(
        _dense_body,
        grid=grid,
        in_specs=[
            pl.BlockSpec((BN_ROWS, D), lambda i: (i, 0)),
            pl.BlockSpec((NC, BN_ROWS, D), lambda i: (0, i, 0)),
            pl.BlockSpec((BN_ROWS, NW), lambda i: (i, 0)),
            pl.BlockSpec((D, D), lambda i: (0, 0)),
            pl.BlockSpec((D, D), lambda i: (0, 0)),
            pl.BlockSpec((1, D), lambda i: (0, 0)),
            pl.BlockSpec((1, D), lambda i: (0, 0)),
        ],
        out_specs=pl.BlockSpec((BN_ROWS, D), lambda i: (i, 0)),
        out_shape=jax.ShapeDtypeStruct((N, D), jnp.float32),
    )(x, aggp, degT, WlT, WrT, s, t)


def kernel(edge_index, emb, Wl0, bl0, Wr0, g0, b0, rm0, rv0,
           Wl1, bl1, Wr1, g1, b1, rm1, rv1):
    src = edge_index[0]
    dst = edge_index[1]

    s0 = (g0 / jnp.sqrt(rv0 + EPS)).reshape(1, D)
    t0 = ((bl0 - rm0) * s0[0] + b0).reshape(1, D)
    s1 = (g1 / jnp.sqrt(rv1 + EPS)).reshape(1, D)
    t1 = ((bl1 - rm1) * s1[0] + b1).reshape(1, D)

    aggp0, degp = _sc_agg_deg(src, dst, emb)
    degT = degp.reshape(NW, N).T
    x1 = _dense(emb, aggp0, degT, Wl0.T, Wr0.T, s0, t0)
    (aggp1,) = _sc_agg(src, dst, x1)
    x2 = _dense(x1, aggp1, degT, Wl1.T, Wr1.T, s1, t1)
    return x2
